# baseline (speedup 1.0000x reference)
import jax
import jax.numpy as jnp
from jax import lax
from jax.experimental import pallas as pl
from jax.experimental.pallas import tpu as pltpu
from jax.experimental.pallas import tpu_sc as plsc

NUM_BUCKETS = 1000000
DIM = 64
B_ROWS = 4
SEQ = 8192
TOTAL = B_ROWS * SEQ

_info = plsc.get_sparse_core_info()
NC, NS, L = _info.num_cores, _info.num_subcores, _info.num_lanes
NW = NC * NS
B_PER_W = TOTAL // NW
PASS_ROWS = 512
N_PASS = B_PER_W // PASS_ROWS
GCHUNK = 128
N_G = PASS_ROWS // GCHUNK


def _sc_kernel(ids_hbm, table_hbm, out_hbm, ext_v, idx_v, rows_v, stage_v, sem):
    wid = lax.axis_index("s") * NC + lax.axis_index("c")
    base = wid * B_PER_W

    pltpu.sync_copy(ids_hbm.at[pl.ds(base, B_PER_W + 8)], ext_v)

    lane = lax.iota(jnp.int32, 16)

    def make_hash_step(p):
        def hash_step(s, _):
            i0 = s * 16
            cur = ext_v[pl.ds(i0 + 8, 16)]
            prev = ext_v[pl.ds(i0 + 7, 16)]
            prev = prev * jnp.minimum((base + i0 + lane) & (SEQ - 1), 1)
            h = (prev * 31 + cur) % NUM_BUCKETS
            idx_v[pl.ds(i0 - p * PASS_ROWS, 16)] = h
            return 0

        return hash_step

    for p in range(N_PASS):
        lax.fori_loop(
            p * (PASS_ROWS // 16),
            (p + 1) * (PASS_ROWS // 16),
            make_hash_step(p),
            0,
            unroll=8,
        )

        copies = []
        for g in range(N_G):
            copies.append(
                pltpu.async_copy(
                    table_hbm.at[idx_v.at[pl.ds(g * GCHUNK, GCHUNK)]],
                    rows_v.at[pl.ds(g * GCHUNK, GCHUNK)],
                    sem,
                )
            )
        for c in copies:
            c.wait()

        def repitch(j, _):
            for c in range(DIM // 16):
                stage_v[j, pl.ds(c * 16, 16)] = rows_v[j, pl.ds(c * 16, 16)]
            return 0

        lax.fori_loop(0, PASS_ROWS, repitch, 0, unroll=4)

        pltpu.sync_copy(
            stage_v, out_hbm.at[pl.ds(base + p * PASS_ROWS, PASS_ROWS)]
        )


@jax.jit
def kernel(input_ids, emb_weight):
    ids_flat = input_ids.reshape(-1).astype(jnp.int32)
    ids_pad = jnp.concatenate([jnp.zeros((8,), jnp.int32), ids_flat])

    mesh = plsc.VectorSubcoreMesh(core_axis_name="c", subcore_axis_name="s")
    out = pl.kernel(
        _sc_kernel,
        mesh=mesh,
        out_type=jax.ShapeDtypeStruct((TOTAL, 2 * DIM), jnp.float32),
        scratch_types=[
            pltpu.VMEM((B_PER_W + 8,), jnp.int32),
            pltpu.VMEM((PASS_ROWS,), jnp.int32),
            pltpu.VMEM((PASS_ROWS, DIM), jnp.float32),
            pltpu.VMEM((PASS_ROWS, 2 * DIM), jnp.float32),
            pltpu.SemaphoreType.DMA,
        ],
        compiler_params=pltpu.CompilerParams(
            use_tc_tiling_on_sc=False, needs_layout_passes=False
        ),
    )(ids_pad, emb_weight)
    return out[:, :DIM].reshape(B_ROWS, SEQ, DIM)
